# Initial kernel scaffold; baseline (speedup 1.0000x reference)
import functools

import jax
import jax.numpy as jnp
from jax import lax
from jax.experimental import pallas as pl
from jax.experimental.pallas import tpu as pltpu
from jax.experimental.pallas import tpu_sc as plsc

B = 4096
F = 26
V = 100000
D = 32
NUM = 13

NC = 2
NS = 16
NW = NC * NS

N = F * B
N_PER_W = N // NW
CHUNK = 128
C = N_PER_W // CHUNK

_sc_mesh = plsc.VectorSubcoreMesh(core_axis_name="c", subcore_axis_name="s")


@functools.partial(
    pl.kernel,
    mesh=_sc_mesh,
    out_type=jax.ShapeDtypeStruct((N, D), jnp.float32),
    scratch_types=[
        pltpu.VMEM((C, CHUNK), jnp.int32),
        pltpu.VMEM((N_PER_W, D), jnp.float32),
        pltpu.SemaphoreType.DMA,
    ],
)
def _sc_gather(table_hbm, idx_hbm, out_hbm, idx_v, rows_v, sem):
    w = lax.axis_index("s") * NC + lax.axis_index("c")
    pltpu.sync_copy(idx_hbm.at[pl.ds(w * C, C)], idx_v)
    copies = []
    for j in range(C):
        copies.append(
            pltpu.async_copy(
                table_hbm.at[idx_v.at[j]],
                rows_v.at[pl.ds(j * CHUNK, CHUNK)],
                sem,
            )
        )
    for cp in copies:
        cp.wait()
    pltpu.sync_copy(rows_v, out_hbm.at[pl.ds(w * N_PER_W, N_PER_W)])


BB = 512


def _mlp_body(emb_ref, num_ref, w1a_ref, w1b_ref, b1_ref, w2_ref, b2_ref,
              w3_ref, b3_ref, w4_ref, b4_ref, out_ref):
    h = jnp.dot(emb_ref[...], w1a_ref[...], preferred_element_type=jnp.float32)
    h += jnp.dot(num_ref[...], w1b_ref[...], preferred_element_type=jnp.float32)
    h = jnp.maximum(h + b1_ref[...], 0.0)
    h = jnp.dot(h, w2_ref[...], preferred_element_type=jnp.float32)
    h = jnp.maximum(h + b2_ref[...], 0.0)
    h = jnp.dot(h, w3_ref[...], preferred_element_type=jnp.float32)
    h = jnp.maximum(h + b3_ref[...], 0.0)
    out_ref[...] = (
        jnp.dot(h, w4_ref[...], preferred_element_type=jnp.float32) + b4_ref[...]
    )


def _mlp(emb, num, w1a, w1b, b1, w2, b2, w3, b3, w4, b4):
    grid = B // BB
    full = lambda i: (0, 0)
    return pl.pallas_call(
        _mlp_body,
        grid=(grid,),
        in_specs=[
            pl.BlockSpec((BB, F * D), lambda i: (i, 0)),
            pl.BlockSpec((BB, NUM), lambda i: (i, 0)),
            pl.BlockSpec((F * D, 512), full),
            pl.BlockSpec((NUM, 512), full),
            pl.BlockSpec((1, 512), full),
            pl.BlockSpec((512, 256), full),
            pl.BlockSpec((1, 256), full),
            pl.BlockSpec((256, 128), full),
            pl.BlockSpec((1, 128), full),
            pl.BlockSpec((128, 1), full),
            pl.BlockSpec((1, 1), full),
        ],
        out_specs=pl.BlockSpec((BB, 1), lambda i: (i, 0)),
        out_shape=jax.ShapeDtypeStruct((B, 1), jnp.float32),
    )(emb, num, w1a, w1b, b1, w2, b2, w3, b3, w4, b4)


def kernel(numerical_features, cat_features, tables, W1, b1, W2, b2, W3, b3,
           W4, b4):
    table_flat = tables.reshape(F * V, D)
    offs = jnp.arange(F, dtype=jnp.int32) * V
    idx = cat_features[:, :, 0].T + offs[None, :]
    idx2d = idx.reshape(NW * C, CHUNK)
    emb = _sc_gather(table_flat, idx2d)
    emb2 = emb.reshape(B, F * D)
    return _mlp(
        emb2,
        numerical_features,
        W1[: F * D],
        W1[F * D :],
        b1.reshape(1, -1),
        W2,
        b2.reshape(1, -1),
        W3,
        b3.reshape(1, -1),
        W4,
        b4.reshape(1, 1),
    )


# speedup vs baseline: 2.2096x; 2.2096x over previous
import functools

import jax
import jax.numpy as jnp
from jax import lax
from jax.experimental import pallas as pl
from jax.experimental.pallas import tpu as pltpu
from jax.experimental.pallas import tpu_sc as plsc

B = 4096
F = 26
V = 100000
D = 32
NUM = 13

NC = 2
NS = 16
NW = NC * NS

N = F * B
N_PER_W = N // NW
CHUNK = 128
C = N_PER_W // CHUNK

@functools.lru_cache(maxsize=None)
def _make_sc_gather():
    mesh = plsc.VectorSubcoreMesh(core_axis_name="c", subcore_axis_name="s")

    @functools.partial(
        pl.kernel,
        mesh=mesh,
        out_type=jax.ShapeDtypeStruct((N, D), jnp.float32),
        scratch_types=[
            pltpu.VMEM((C, CHUNK), jnp.int32),
            pltpu.VMEM((N_PER_W, D), jnp.float32),
            pltpu.SemaphoreType.DMA,
        ],
        compiler_params=pltpu.CompilerParams(use_tc_tiling_on_sc=False),
    )
    def _sc_gather(table_hbm, idx_hbm, out_hbm, idx_v, rows_v, sem):
        w = lax.axis_index("s") * NC + lax.axis_index("c")
        pltpu.sync_copy(idx_hbm.at[w], idx_v)
        copies = []
        for j in range(C):
            copies.append(
                pltpu.async_copy(
                    table_hbm.at[idx_v.at[j]],
                    rows_v.at[pl.ds(j * CHUNK, CHUNK)],
                    sem,
                )
            )
        for cp in copies:
            cp.wait()
        pltpu.sync_copy(rows_v, out_hbm.at[pl.ds(w * N_PER_W, N_PER_W)])

    return _sc_gather


BB = 512


def _mlp_body(emb_ref, num_ref, w1a_ref, w1b_ref, b1_ref, w2_ref, b2_ref,
              w3_ref, b3_ref, w4_ref, b4_ref, out_ref):
    h = jnp.dot(emb_ref[...], w1a_ref[...], preferred_element_type=jnp.float32)
    h += jnp.dot(num_ref[...], w1b_ref[...], preferred_element_type=jnp.float32)
    h = jnp.maximum(h + b1_ref[...], 0.0)
    h = jnp.dot(h, w2_ref[...], preferred_element_type=jnp.float32)
    h = jnp.maximum(h + b2_ref[...], 0.0)
    h = jnp.dot(h, w3_ref[...], preferred_element_type=jnp.float32)
    h = jnp.maximum(h + b3_ref[...], 0.0)
    out_ref[...] = (
        jnp.dot(h, w4_ref[...], preferred_element_type=jnp.float32) + b4_ref[...]
    )


def _mlp(emb, num, w1a, w1b, b1, w2, b2, w3, b3, w4, b4):
    grid = B // BB
    full = lambda i: (0, 0)
    return pl.pallas_call(
        _mlp_body,
        grid=(grid,),
        in_specs=[
            pl.BlockSpec((BB, F * D), lambda i: (i, 0)),
            pl.BlockSpec((BB, NUM), lambda i: (i, 0)),
            pl.BlockSpec((F * D, 512), full),
            pl.BlockSpec((NUM, 512), full),
            pl.BlockSpec((1, 512), full),
            pl.BlockSpec((512, 256), full),
            pl.BlockSpec((1, 256), full),
            pl.BlockSpec((256, 128), full),
            pl.BlockSpec((1, 128), full),
            pl.BlockSpec((128, 1), full),
            pl.BlockSpec((1, 1), full),
        ],
        out_specs=pl.BlockSpec((BB, 1), lambda i: (i, 0)),
        out_shape=jax.ShapeDtypeStruct((B, 1), jnp.float32),
    )(emb, num, w1a, w1b, b1, w2, b2, w3, b3, w4, b4)


def kernel(numerical_features, cat_features, tables, W1, b1, W2, b2, W3, b3,
           W4, b4):
    table_flat = tables.reshape(F * V, D)
    offs = jnp.arange(F, dtype=jnp.int32) * V
    idx = cat_features[:, :, 0].T + offs[None, :]
    idx3d = idx.reshape(NW, C, CHUNK)
    emb = _make_sc_gather()(table_flat, idx3d)
    emb2 = emb.reshape(B, F * D)
    return _mlp(
        emb2,
        numerical_features,
        W1[: F * D],
        W1[F * D :],
        b1.reshape(1, -1),
        W2,
        b2.reshape(1, -1),
        W3,
        b3.reshape(1, -1),
        W4,
        b4.reshape(1, 1),
    )
